# baseline (speedup 1.0000x reference)
import dataclasses
import functools

import jax
import jax.numpy as jnp
from jax import lax
from jax.experimental import pallas as pl
from jax.experimental.pallas import tpu as pltpu
from jax.experimental.pallas import tpu_sc as plsc

L = 16
NC = 2
NS = 16
NW = NC * NS
BATCH = 32
S_OUT = 10000
SEG_PER = 320
SEG_TOT = NW * SEG_PER


def _sload(ref, idx):
    return ref[pl.ds(idx, L)][0]


def _sc_body(x_hbm, ids_hbm, wb_hbm, rowb_hbm, out_hbm, wout_hbm,
             xs, ids_v, ws, wb_v, rowb_v, acc, cnt):
    n, d = x_hbm.shape
    nchunk = d // L
    cid = lax.axis_index("c")
    sid = lax.axis_index("s")
    wid = cid * NS + sid

    pltpu.sync_copy(wb_hbm, wb_v.at[pl.ds(0, d + 8)])
    pltpu.sync_copy(rowb_hbm, rowb_v.at[pl.ds(0, NW + 8)])
    bias = _sload(wb_v, d)
    zero16 = jnp.zeros((L,), jnp.float32)
    lane = lax.iota(jnp.int32, L)
    w_chunks = [wb_v[pl.ds(c * L, L)] for c in range(nchunk)]

    @pl.loop(0, SEG_PER)
    def _zero(s):
        for c in range(nchunk):
            acc[s, pl.ds(c * L, L)] = zero16
        cnt[s, :] = zero16

    rstart = _sload(rowb_v, wid)
    rend = _sload(rowb_v, wid + 1)
    seg_lo = wid * SEG_PER
    b0 = rstart // BATCH
    b1 = (rend + BATCH - 1) // BATCH

    def batch_body(bi, carry):
        base = bi * BATCH
        pltpu.sync_copy(x_hbm.at[pl.ds(base, BATCH), :], xs)
        pltpu.sync_copy(ids_hbm.at[pl.ds(base, BATCH)], ids_v.at[pl.ds(0, BATCH)])

        for g in range(BATCH // L):
            def row_dot(k, tvec, g=g):
                i = g * L + k
                dvec = zero16
                for c in range(nchunk):
                    dvec = dvec + xs[i, pl.ds(c * L, L)] * w_chunks[c]
                t = jnp.sum(dvec) + bias
                return jnp.where(lane == k, t, tvec)

            tvec = lax.fori_loop(0, L, row_dot, zero16)
            wv = 1.0 / (1.0 + jnp.exp(-tvec))
            ws[pl.ds(g * L, L)] = wv

        @pl.when(base >= rstart)
        def _own():
            pltpu.sync_copy(ws.at[pl.ds(0, BATCH)], wout_hbm.at[pl.ds(base, BATCH)])

        i_lo = jnp.maximum(rstart - base, 0)
        i_hi = jnp.minimum(rend - base, BATCH)

        def acc_row(i, carry2):
            ls = _sload(ids_v, i) - seg_lo
            wrow = jnp.full((L,), _sload(ws, i))
            for c in range(nchunk):
                acc[ls, pl.ds(c * L, L)] += wrow * xs[i, pl.ds(c * L, L)]
            cnt[ls, :] += 1.0
            return carry2

        lax.fori_loop(i_lo, i_hi, acc_row, 0)
        return carry

    lax.fori_loop(b0, b1, batch_body, 0)

    @pl.loop(0, SEG_PER)
    def _div(s):
        inv = 1.0 / cnt[s, :]
        for c in range(nchunk):
            acc[s, pl.ds(c * L, L)] = acc[s, pl.ds(c * L, L)] * inv

    pltpu.sync_copy(acc, out_hbm.at[pl.ds(seg_lo, SEG_PER), :])


def kernel(x, supercase_indexes, W, b):
    n, d = x.shape
    ids = supercase_indexes.astype(jnp.int32)
    wb = jnp.concatenate(
        [W.reshape(-1).astype(jnp.float32), b.reshape(-1).astype(jnp.float32),
         jnp.zeros((7,), jnp.float32)])
    qs = jnp.arange(NW + 1, dtype=jnp.int32) * SEG_PER
    rowb = jnp.searchsorted(ids, qs).astype(jnp.int32)
    rowb = jnp.concatenate([rowb, jnp.zeros((7,), jnp.int32)])

    mesh = plsc.VectorSubcoreMesh(core_axis_name="c", subcore_axis_name="s")
    cp = pltpu.CompilerParams()
    if "needs_layout_passes" in pltpu.CompilerParams.__dataclass_fields__:
        cp = dataclasses.replace(cp, needs_layout_passes=False)
    if "use_tc_tiling_on_sc" in pltpu.CompilerParams.__dataclass_fields__:
        cp = dataclasses.replace(cp, use_tc_tiling_on_sc=False)
    run = pl.kernel(
        _sc_body,
        compiler_params=cp,
        out_type=[
            jax.ShapeDtypeStruct((SEG_TOT, d), jnp.float32),
            jax.ShapeDtypeStruct((n,), jnp.float32),
        ],
        mesh=mesh,
        scratch_types=[
            pltpu.VMEM((BATCH, d), jnp.float32),
            pltpu.VMEM((BATCH + L,), jnp.int32),
            pltpu.VMEM((BATCH + L,), jnp.float32),
            pltpu.VMEM((d + 8 + L,), jnp.float32),
            pltpu.VMEM((NW + 8 + L,), jnp.int32),
            pltpu.VMEM((SEG_PER, d), jnp.float32),
            pltpu.VMEM((SEG_PER, L), jnp.float32),
        ],
    )
    out_pad, wout = run(x, ids, wb, rowb)
    return out_pad[:S_OUT], wout.reshape(n, 1)
